# Initial kernel scaffold; baseline (speedup 1.0000x reference)
import jax
import jax.numpy as jnp
from jax.experimental import pallas as pl


def kernel(idx, tables):
    raise NotImplementedError("write your pallas kernel here")


# speedup vs baseline: 20.2722x; 20.2722x over previous
import functools

import jax
import jax.numpy as jnp
from jax import lax
from jax.experimental import pallas as pl
from jax.experimental.pallas import tpu as pltpu
from jax.experimental.pallas import tpu_sc as plsc

_VOCAB = 100000
_CHAR_DIM = 32
_N_COMP = 4
_NC = 2
_NS = 16
_NW = _NC * _NS
_T = 128


def _build_gather(ntok: int):
    tpw = ntok // _NW
    n_chunks = tpw // _T
    mesh = plsc.VectorSubcoreMesh(core_axis_name="c", subcore_axis_name="s")

    @functools.partial(
        pl.kernel,
        out_type=jax.ShapeDtypeStruct((ntok, _N_COMP * _CHAR_DIM), jnp.float32),
        mesh=mesh,
        scratch_types=[
            pltpu.VMEM((_T,), jnp.int32),
            pltpu.VMEM((_T, _N_COMP * _CHAR_DIM), jnp.float32),
            pltpu.SemaphoreType.DMA,
        ],
    )
    def gather(idx_hbm, tab_hbm, out_hbm, idx_v, rows_v, sem):
        wid = lax.axis_index("s") * _NC + lax.axis_index("c")
        wbase = wid * tpw

        @pl.loop(0, n_chunks)
        def _chunk(j):
            base = wbase + j * _T
            pltpu.sync_copy(idx_hbm.at[pl.ds(base, _T)], idx_v)
            pltpu.async_copy(tab_hbm.at[idx_v], rows_v, sem).wait()
            pltpu.sync_copy(rows_v, out_hbm.at[pl.ds(base, _T)])

    return gather


def kernel(idx, tables):
    b, s = idx.shape
    ntok = b * s
    idx_flat = idx.reshape(ntok).astype(jnp.int32)
    ftab = jnp.transpose(tables, (1, 0, 2)).reshape(_VOCAB, _N_COMP * _CHAR_DIM)
    out = _build_gather(ntok)(idx_flat, ftab)
    return out.reshape(b, s, _N_COMP * _CHAR_DIM)


# speedup vs baseline: 34.2000x; 1.6870x over previous
import functools

import jax
import jax.numpy as jnp
from jax import lax
from jax.experimental import pallas as pl
from jax.experimental.pallas import tpu as pltpu
from jax.experimental.pallas import tpu_sc as plsc

_VOCAB = 100000
_CHAR_DIM = 32
_N_COMP = 4
_NC = 2
_NS = 16
_NW = _NC * _NS
_T = 128
_NBUF = 4


def _build_gather(ntok: int):
    tpw = ntok // _NW
    n_chunks = tpw // _T
    mesh = plsc.VectorSubcoreMesh(core_axis_name="c", subcore_axis_name="s")

    @functools.partial(
        pl.kernel,
        out_type=jax.ShapeDtypeStruct((ntok, _N_COMP * _CHAR_DIM), jnp.float32),
        mesh=mesh,
        scratch_types=[
            pltpu.VMEM((_NBUF, _T), jnp.int32),
            pltpu.VMEM((_NBUF, _T, _N_COMP * _CHAR_DIM), jnp.float32),
            pltpu.SemaphoreType.DMA,
            pltpu.SemaphoreType.DMA,
        ],
    )
    def gather(idx_hbm, tab_hbm, out_hbm, idx_v, rows_v, gsem, wsem):
        wid = lax.axis_index("s") * _NC + lax.axis_index("c")
        wbase = wid * tpw

        for b in range(_NBUF):
            base = wbase + b * _T
            pltpu.sync_copy(idx_hbm.at[pl.ds(base, _T)], idx_v.at[b])
            pltpu.async_copy(tab_hbm.at[idx_v.at[b]], rows_v.at[b], gsem)

        @pl.loop(0, n_chunks, step=_NBUF)
        def _ring(j):
            for b in range(_NBUF):
                k = j + b
                base = wbase + k * _T
                out_slice = out_hbm.at[pl.ds(base, _T)]
                pltpu.make_async_copy(out_slice, rows_v.at[b], gsem).wait()
                pltpu.async_copy(rows_v.at[b], out_slice, wsem)
                nk = k + _NBUF

                @pl.when(nk < n_chunks)
                def _prefetch_idx():
                    nbase = wbase + nk * _T
                    pltpu.sync_copy(idx_hbm.at[pl.ds(nbase, _T)], idx_v.at[b])

                pltpu.make_async_copy(rows_v.at[b], out_slice, wsem).wait()

                @pl.when(nk < n_chunks)
                def _next_gather():
                    pltpu.async_copy(tab_hbm.at[idx_v.at[b]], rows_v.at[b], gsem)

    return gather


def kernel(idx, tables):
    b, s = idx.shape
    ntok = b * s
    idx_flat = idx.reshape(ntok).astype(jnp.int32)
    ftab = jnp.transpose(tables, (1, 0, 2)).reshape(_VOCAB, _N_COMP * _CHAR_DIM)
    out = _build_gather(ntok)(idx_flat, ftab)
    return out.reshape(b, s, _N_COMP * _CHAR_DIM)


# speedup vs baseline: 34.4795x; 1.0082x over previous
import functools

import jax
import jax.numpy as jnp
from jax import lax
from jax.experimental import pallas as pl
from jax.experimental.pallas import tpu as pltpu
from jax.experimental.pallas import tpu_sc as plsc

_VOCAB = 100000
_CHAR_DIM = 32
_N_COMP = 4
_NC = 2
_NS = 16
_NW = _NC * _NS
_T = 128
_NBUF = 6
_WLAG = 2


def _build_gather(ntok: int):
    tpw = ntok // _NW
    n_chunks = tpw // _T
    mesh = plsc.VectorSubcoreMesh(core_axis_name="c", subcore_axis_name="s")

    @functools.partial(
        pl.kernel,
        out_type=jax.ShapeDtypeStruct((ntok, _N_COMP * _CHAR_DIM), jnp.float32),
        mesh=mesh,
        scratch_types=[
            pltpu.VMEM((n_chunks, _T), jnp.int32),
            pltpu.VMEM((_NBUF, _T, _N_COMP * _CHAR_DIM), jnp.float32),
            pltpu.SemaphoreType.DMA,
            pltpu.SemaphoreType.DMA,
        ],
    )
    def gather(idx_hbm, tab_hbm, out_hbm, idx_v, rows_v, gsem, wsem):
        wid = lax.axis_index("s") * _NC + lax.axis_index("c")
        wbase = wid * tpw

        def out_at(k):
            return out_hbm.at[pl.ds(wbase + k * _T, _T)]

        def fire_gather(k, b):
            pltpu.async_copy(tab_hbm.at[idx_v.at[k]], rows_v.at[b], gsem)

        def wait_gather(k, b):
            pltpu.make_async_copy(out_at(k), rows_v.at[b], gsem).wait()

        def fire_write(k, b):
            pltpu.async_copy(rows_v.at[b], out_at(k), wsem)

        def wait_write(k, b):
            pltpu.make_async_copy(rows_v.at[b], out_at(k), wsem).wait()

        pltpu.sync_copy(idx_hbm.at[pl.ds(wid * n_chunks, n_chunks)], idx_v)

        for b in range(_NBUF):
            fire_gather(b, b)
        for k in range(_WLAG):
            wait_gather(k, k % _NBUF)
            fire_write(k, k % _NBUF)

        @pl.loop(_WLAG, n_chunks - _NBUF, step=_NBUF)
        def _ring(j):
            for i in range(_NBUF):
                k = j + i
                b = (_WLAG + i) % _NBUF
                wait_gather(k, b)
                fire_write(k, b)
                kp, bp = k - _WLAG, (i % _NBUF)
                wait_write(kp, bp)
                fire_gather(kp + _NBUF, bp)

        for k in range(n_chunks - _NBUF, n_chunks):
            wait_gather(k, k % _NBUF)
            fire_write(k, k % _NBUF)
            kp = k - _WLAG
            wait_write(kp, kp % _NBUF)
            if kp + _NBUF < n_chunks:
                fire_gather(kp + _NBUF, kp % _NBUF)
        for k in range(n_chunks - _WLAG, n_chunks):
            wait_write(k, k % _NBUF)

    return gather


def kernel(idx, tables):
    b, s = idx.shape
    ntok = b * s
    idx_rows = idx.reshape(ntok // _T, _T).astype(jnp.int32)
    ftab = jnp.transpose(tables, (1, 0, 2)).reshape(_VOCAB, _N_COMP * _CHAR_DIM)
    out = _build_gather(ntok)(idx_rows, ftab)
    return out.reshape(b, s, _N_COMP * _CHAR_DIM)
